# Initial kernel scaffold; baseline (speedup 1.0000x reference)
import jax
import jax.numpy as jnp
from jax.experimental import pallas as pl


def kernel(h, edge_index, perm_features, W1, b1, W2, b2, Wc, bc):
    raise NotImplementedError("write your pallas kernel here")


# speedup vs baseline: 2.0493x; 2.0493x over previous
import jax
import jax.numpy as jnp
from jax import lax
from jax.experimental import pallas as pl
from jax.experimental.pallas import tpu as pltpu
from jax.experimental.pallas import tpu_sc as plsc

N = 10000
E = 320000
D = 128
NC, NS, L = 2, 16, 16
NW = NC * NS
B = 128
NB = 80
E_PAD = NW * NB * B
R_T = 632
N_ACC = NS * R_T
BM = 1000


def _segsum_body(src_hbm, dst_hbm, tab_hbm, out_hbm, acc, src_v, dst_v, rows, sem):
    cid = lax.axis_index("c")
    sid = lax.axis_index("s")
    wid = cid * NS + sid
    pltpu.sync_copy(src_hbm.at[wid], src_v)
    pltpu.sync_copy(dst_hbm.at[wid], dst_v)
    zero = jnp.zeros((L,), jnp.float32)

    def zrow(i, c):
        for k in range(D // L):
            rows[i, pl.ds(k * L, L)] = zero
        return c

    lax.fori_loop(0, B, zrow, 0)
    r0 = sid * R_T
    for k in range(R_T // B):
        pltpu.sync_copy(rows, acc.at[pl.ds(r0 + k * B, B)])
    rem = R_T - (R_T // B) * B
    pltpu.sync_copy(rows.at[pl.ds(0, rem)], acc.at[pl.ds(r0 + (R_T // B) * B, rem)])
    plsc.subcore_barrier()

    def step(j, c):
        pltpu.async_copy(tab_hbm.at[src_v.at[j]], rows, sem).wait()
        pltpu.sync_copy(rows, acc.at[dst_v.at[j]], add=True)
        return c

    lax.fori_loop(0, NB, step, 0)
    plsc.subcore_barrier()
    pltpu.sync_copy(acc.at[pl.ds(r0, R_T)], out_hbm.at[cid, pl.ds(r0, R_T)])


_segsum = pl.kernel(
    _segsum_body,
    out_type=jax.ShapeDtypeStruct((NC, N_ACC, D), jnp.float32),
    mesh=plsc.VectorSubcoreMesh(core_axis_name="c", subcore_axis_name="s",
                                num_cores=NC, num_subcores=NS),
    scratch_types=[
        pltpu.VMEM_SHARED((N_ACC, D), jnp.float32),
        pltpu.VMEM((NB, B), jnp.int32),
        pltpu.VMEM((NB, B), jnp.int32),
        pltpu.VMEM((B, D), jnp.float32),
        pltpu.SemaphoreType.DMA,
    ],
)


def _prep1_body(d0_ref, d1_ref, h_ref, norm_ref, s0_ref):
    deg = d0_ref[...] + d1_ref[...]
    deg = jnp.where(deg < 1.0, 1.0, deg)
    nrm = lax.rsqrt(deg)
    norm_ref[...] = nrm
    s0_ref[...] = h_ref[...] * nrm


def _prep1(d0, d1, h):
    return pl.pallas_call(
        _prep1_body,
        grid=(N // BM,),
        in_specs=[
            pl.BlockSpec((BM, 1), lambda i: (i, 0)),
            pl.BlockSpec((BM, 1), lambda i: (i, 0)),
            pl.BlockSpec((BM, D), lambda i: (i, 0)),
        ],
        out_specs=[
            pl.BlockSpec((BM, 1), lambda i: (i, 0)),
            pl.BlockSpec((BM, D), lambda i: (i, 0)),
        ],
        out_shape=[
            jax.ShapeDtypeStruct((N, 1), jnp.float32),
            jax.ShapeDtypeStruct((N, D), jnp.float32),
        ],
    )(d0, d1, h)


def _comb_body(pa_ref, pb_ref, norm_ref, out_ref):
    nrm = norm_ref[...]
    out_ref[...] = (nrm * nrm) * (pa_ref[...] + pb_ref[...])


def _comb(pa, pb, norm):
    return pl.pallas_call(
        _comb_body,
        grid=(N // BM,),
        in_specs=[
            pl.BlockSpec((BM, D), lambda i: (i, 0)),
            pl.BlockSpec((BM, D), lambda i: (i, 0)),
            pl.BlockSpec((BM, 1), lambda i: (i, 0)),
        ],
        out_specs=pl.BlockSpec((BM, D), lambda i: (i, 0)),
        out_shape=jax.ShapeDtypeStruct((N, D), jnp.float32),
    )(pa, pb, norm)


def _layer1_body(h_ref, p1a_ref, p1b_ref, p2a_ref, p2b_ref, norm_ref, w_ref,
                 b_ref, y_ref, t0_ref):
    nrm = norm_ref[...]
    x1 = nrm * (p1a_ref[...] + p1b_ref[...])
    x2 = nrm * (p2a_ref[...] + p2b_ref[...])
    acc = jnp.dot(h_ref[...], w_ref[0:D, :], preferred_element_type=jnp.float32)
    acc = acc + jnp.dot(x1, w_ref[D:2 * D, :], preferred_element_type=jnp.float32)
    acc = acc + jnp.dot(x2, w_ref[2 * D:3 * D, :], preferred_element_type=jnp.float32)
    y = jnp.maximum(acc + b_ref[...], 0.0)
    y_ref[...] = y
    t0_ref[...] = y * nrm


def _layer1(h, p1a, p1b, p2a, p2b, norm, W1, b1):
    return pl.pallas_call(
        _layer1_body,
        grid=(N // BM,),
        in_specs=[
            pl.BlockSpec((BM, D), lambda i: (i, 0)),
            pl.BlockSpec((BM, D), lambda i: (i, 0)),
            pl.BlockSpec((BM, D), lambda i: (i, 0)),
            pl.BlockSpec((BM, D), lambda i: (i, 0)),
            pl.BlockSpec((BM, D), lambda i: (i, 0)),
            pl.BlockSpec((BM, 1), lambda i: (i, 0)),
            pl.BlockSpec((3 * D, D), lambda i: (0, 0)),
            pl.BlockSpec((1, D), lambda i: (0, 0)),
        ],
        out_specs=[
            pl.BlockSpec((BM, D), lambda i: (i, 0)),
            pl.BlockSpec((BM, D), lambda i: (i, 0)),
        ],
        out_shape=[
            jax.ShapeDtypeStruct((N, D), jnp.float32),
            jax.ShapeDtypeStruct((N, D), jnp.float32),
        ],
    )(h, p1a, p1b, p2a, p2b, norm, W1, b1)


def _final_body(y_ref, q1a_ref, q1b_ref, q2a_ref, q2b_ref, norm_ref, perm_ref,
                w2_ref, b2_ref, wc_ref, bc_ref, out_ref, acc_ref):
    i = pl.program_id(0)

    @pl.when(i == 0)
    def _():
        acc_ref[...] = jnp.zeros_like(acc_ref)

    nrm = norm_ref[...]
    y = y_ref[...]
    y1 = nrm * (q1a_ref[...] + q1b_ref[...])
    y2 = nrm * (q2a_ref[...] + q2b_ref[...])
    acc_ref[...] += jnp.concatenate(
        [jnp.sum(y, axis=0, keepdims=True),
         jnp.sum(y1, axis=0, keepdims=True),
         jnp.sum(y2, axis=0, keepdims=True)], axis=1)

    @pl.when(i == pl.num_programs(0) - 1)
    def _():
        hg = jnp.dot(acc_ref[...] * jnp.float32(1.0 / N), w2_ref[...],
                     preferred_element_type=jnp.float32) + b2_ref[...]
        comb = jnp.concatenate([hg, perm_ref[...]], axis=1)
        out_ref[...] = jnp.dot(comb, wc_ref[...],
                               preferred_element_type=jnp.float32) + bc_ref[...]


def _final(y, q1a, q1b, q2a, q2b, norm, perm, W2, b2, Wc, bc):
    nclass = Wc.shape[1]
    return pl.pallas_call(
        _final_body,
        grid=(N // BM,),
        in_specs=[
            pl.BlockSpec((BM, D), lambda i: (i, 0)),
            pl.BlockSpec((BM, D), lambda i: (i, 0)),
            pl.BlockSpec((BM, D), lambda i: (i, 0)),
            pl.BlockSpec((BM, D), lambda i: (i, 0)),
            pl.BlockSpec((BM, D), lambda i: (i, 0)),
            pl.BlockSpec((BM, 1), lambda i: (i, 0)),
            pl.BlockSpec(perm.shape, lambda i: (0, 0)),
            pl.BlockSpec((3 * D, D), lambda i: (0, 0)),
            pl.BlockSpec((1, D), lambda i: (0, 0)),
            pl.BlockSpec(Wc.shape, lambda i: (0, 0)),
            pl.BlockSpec((1, nclass), lambda i: (0, 0)),
        ],
        out_specs=pl.BlockSpec((1, nclass), lambda i: (0, 0)),
        out_shape=jax.ShapeDtypeStruct((1, nclass), jnp.float32),
        scratch_shapes=[pltpu.VMEM((1, 3 * D), jnp.float32)],
    )(y, q1a, q1b, q2a, q2b, norm, perm, W2, b2, Wc, bc)


def kernel(h, edge_index, perm_features, W1, b1, W2, b2, Wc, bc):
    src = edge_index[0].astype(jnp.int32)
    dst = edge_index[1].astype(jnp.int32)
    pad = E_PAD - E
    src3 = jnp.concatenate([src, jnp.zeros((pad,), jnp.int32)]).reshape(NW, NB, B)
    dst3 = jnp.concatenate([dst, jnp.full((pad,), N, jnp.int32)]).reshape(NW, NB, B)

    degp = _segsum(src3, dst3, jnp.ones((N, D), jnp.float32))
    d0 = degp[0, :N, 0:1]
    d1 = degp[1, :N, 0:1]
    norm, s0 = _prep1(d0, d1, h)

    p1 = _segsum(src3, dst3, s0)
    s1 = _comb(p1[0, :N], p1[1, :N], norm)
    p2 = _segsum(src3, dst3, s1)
    y, t0 = _layer1(h, p1[0, :N], p1[1, :N], p2[0, :N], p2[1, :N], norm, W1,
                    b1.reshape(1, D))

    q1 = _segsum(src3, dst3, t0)
    t1 = _comb(q1[0, :N], q1[1, :N], norm)
    q2 = _segsum(src3, dst3, t1)

    return _final(y, q1[0, :N], q1[1, :N], q2[0, :N], q2[1, :N], norm,
                  perm_features, W2, b2.reshape(1, D), Wc, bc.reshape(1, -1))


# speedup vs baseline: 2.2565x; 1.1011x over previous
import jax
import jax.numpy as jnp
from jax import lax
from jax.experimental import pallas as pl
from jax.experimental.pallas import tpu as pltpu
from jax.experimental.pallas import tpu_sc as plsc

N = 10000
E = 320000
D = 128
NC, NS, L = 2, 16, 16
NW = NC * NS
B = 128
NB = 80
E_PAD = NW * NB * B
R_T = 640
N_ACC = NS * R_T
BM = 1000


CH = 16


def _segsum_body(src_hbm, dst_hbm, tab_hbm, out_hbm, acc, src_c, dst_c,
                 rows0, rows1, sem0, sem1):
    cid = lax.axis_index("c")
    sid = lax.axis_index("s")
    wid = cid * NS + sid
    zero = jnp.zeros((L,), jnp.float32)

    def zrow(i, c):
        for k in range(D // L):
            rows0[i, pl.ds(k * L, L)] = zero
        return c

    lax.fori_loop(0, B, zrow, 0)
    r0 = sid * R_T
    for k in range(R_T // B):
        pltpu.sync_copy(rows0, acc.at[pl.ds(r0 + k * B, B)])
    plsc.subcore_barrier()

    bufs = (rows0, rows1)
    sems = (sem0, sem1)

    def chunk(c, carry):
        pltpu.sync_copy(src_hbm.at[wid, pl.ds(c * CH, CH)], src_c)
        pltpu.sync_copy(dst_hbm.at[wid, pl.ds(c * CH, CH)], dst_c)
        pltpu.async_copy(tab_hbm.at[src_c.at[0]], rows0, sem0)
        pltpu.async_copy(tab_hbm.at[src_c.at[1]], rows1, sem1)

        def pair(g, cc):
            for b in range(2):
                j = 2 * g + b
                rows, sem = bufs[b], sems[b]
                pltpu.make_async_copy(tab_hbm.at[src_c.at[j]], rows, sem).wait()
                pltpu.sync_copy(rows, acc.at[dst_c.at[j]], add=True)

                @pl.when(j + 2 < CH)
                def _():
                    pltpu.async_copy(tab_hbm.at[src_c.at[j + 2]], rows, sem)
            return cc

        lax.fori_loop(0, CH // 2, pair, 0)
        return carry

    lax.fori_loop(0, NB // CH, chunk, 0)
    plsc.subcore_barrier()
    pltpu.sync_copy(acc.at[pl.ds(r0, R_T)], out_hbm.at[cid, pl.ds(r0, R_T)])


_segsum = pl.kernel(
    _segsum_body,
    out_type=jax.ShapeDtypeStruct((NC, N_ACC, D), jnp.float32),
    mesh=plsc.VectorSubcoreMesh(core_axis_name="c", subcore_axis_name="s",
                                num_cores=NC, num_subcores=NS),
    scratch_types=[
        pltpu.VMEM_SHARED((N_ACC, D), jnp.float32),
        pltpu.VMEM((CH, B), jnp.int32),
        pltpu.VMEM((CH, B), jnp.int32),
        pltpu.VMEM((B, D), jnp.float32),
        pltpu.VMEM((B, D), jnp.float32),
        pltpu.SemaphoreType.DMA,
        pltpu.SemaphoreType.DMA,
    ],
)


def _prep1_body(d0_ref, d1_ref, h_ref, norm_ref, s0_ref):
    deg = d0_ref[...] + d1_ref[...]
    deg = jnp.where(deg < 1.0, 1.0, deg)
    nrm = lax.rsqrt(deg)
    norm_ref[...] = nrm
    s0_ref[...] = h_ref[...] * nrm


def _prep1(d0, d1, h):
    return pl.pallas_call(
        _prep1_body,
        grid=(N // BM,),
        in_specs=[
            pl.BlockSpec((BM, 1), lambda i: (i, 0)),
            pl.BlockSpec((BM, 1), lambda i: (i, 0)),
            pl.BlockSpec((BM, D), lambda i: (i, 0)),
        ],
        out_specs=[
            pl.BlockSpec((BM, 1), lambda i: (i, 0)),
            pl.BlockSpec((BM, D), lambda i: (i, 0)),
        ],
        out_shape=[
            jax.ShapeDtypeStruct((N, 1), jnp.float32),
            jax.ShapeDtypeStruct((N, D), jnp.float32),
        ],
    )(d0, d1, h)


def _comb_body(pa_ref, pb_ref, norm_ref, out_ref):
    nrm = norm_ref[...]
    out_ref[...] = (nrm * nrm) * (pa_ref[...] + pb_ref[...])


def _comb(pa, pb, norm):
    return pl.pallas_call(
        _comb_body,
        grid=(N // BM,),
        in_specs=[
            pl.BlockSpec((BM, D), lambda i: (i, 0)),
            pl.BlockSpec((BM, D), lambda i: (i, 0)),
            pl.BlockSpec((BM, 1), lambda i: (i, 0)),
        ],
        out_specs=pl.BlockSpec((BM, D), lambda i: (i, 0)),
        out_shape=jax.ShapeDtypeStruct((N, D), jnp.float32),
    )(pa, pb, norm)


def _layer1_body(h_ref, p1a_ref, p1b_ref, p2a_ref, p2b_ref, norm_ref, w_ref,
                 b_ref, y_ref, t0_ref):
    nrm = norm_ref[...]
    x1 = nrm * (p1a_ref[...] + p1b_ref[...])
    x2 = nrm * (p2a_ref[...] + p2b_ref[...])
    acc = jnp.dot(h_ref[...], w_ref[0:D, :], preferred_element_type=jnp.float32)
    acc = acc + jnp.dot(x1, w_ref[D:2 * D, :], preferred_element_type=jnp.float32)
    acc = acc + jnp.dot(x2, w_ref[2 * D:3 * D, :], preferred_element_type=jnp.float32)
    y = jnp.maximum(acc + b_ref[...], 0.0)
    y_ref[...] = y
    t0_ref[...] = y * nrm


def _layer1(h, p1a, p1b, p2a, p2b, norm, W1, b1):
    return pl.pallas_call(
        _layer1_body,
        grid=(N // BM,),
        in_specs=[
            pl.BlockSpec((BM, D), lambda i: (i, 0)),
            pl.BlockSpec((BM, D), lambda i: (i, 0)),
            pl.BlockSpec((BM, D), lambda i: (i, 0)),
            pl.BlockSpec((BM, D), lambda i: (i, 0)),
            pl.BlockSpec((BM, D), lambda i: (i, 0)),
            pl.BlockSpec((BM, 1), lambda i: (i, 0)),
            pl.BlockSpec((3 * D, D), lambda i: (0, 0)),
            pl.BlockSpec((1, D), lambda i: (0, 0)),
        ],
        out_specs=[
            pl.BlockSpec((BM, D), lambda i: (i, 0)),
            pl.BlockSpec((BM, D), lambda i: (i, 0)),
        ],
        out_shape=[
            jax.ShapeDtypeStruct((N, D), jnp.float32),
            jax.ShapeDtypeStruct((N, D), jnp.float32),
        ],
    )(h, p1a, p1b, p2a, p2b, norm, W1, b1)


def _final_body(y_ref, q1a_ref, q1b_ref, q2a_ref, q2b_ref, norm_ref, perm_ref,
                w2_ref, b2_ref, wc_ref, bc_ref, out_ref, acc_ref):
    i = pl.program_id(0)

    @pl.when(i == 0)
    def _():
        acc_ref[...] = jnp.zeros_like(acc_ref)

    nrm = norm_ref[...]
    y = y_ref[...]
    y1 = nrm * (q1a_ref[...] + q1b_ref[...])
    y2 = nrm * (q2a_ref[...] + q2b_ref[...])
    acc_ref[...] += jnp.concatenate(
        [jnp.sum(y, axis=0, keepdims=True),
         jnp.sum(y1, axis=0, keepdims=True),
         jnp.sum(y2, axis=0, keepdims=True)], axis=1)

    @pl.when(i == pl.num_programs(0) - 1)
    def _():
        hg = jnp.dot(acc_ref[...] * jnp.float32(1.0 / N), w2_ref[...],
                     preferred_element_type=jnp.float32) + b2_ref[...]
        comb = jnp.concatenate([hg, perm_ref[...]], axis=1)
        out_ref[...] = jnp.dot(comb, wc_ref[...],
                               preferred_element_type=jnp.float32) + bc_ref[...]


def _final(y, q1a, q1b, q2a, q2b, norm, perm, W2, b2, Wc, bc):
    nclass = Wc.shape[1]
    return pl.pallas_call(
        _final_body,
        grid=(N // BM,),
        in_specs=[
            pl.BlockSpec((BM, D), lambda i: (i, 0)),
            pl.BlockSpec((BM, D), lambda i: (i, 0)),
            pl.BlockSpec((BM, D), lambda i: (i, 0)),
            pl.BlockSpec((BM, D), lambda i: (i, 0)),
            pl.BlockSpec((BM, D), lambda i: (i, 0)),
            pl.BlockSpec((BM, 1), lambda i: (i, 0)),
            pl.BlockSpec(perm.shape, lambda i: (0, 0)),
            pl.BlockSpec((3 * D, D), lambda i: (0, 0)),
            pl.BlockSpec((1, D), lambda i: (0, 0)),
            pl.BlockSpec(Wc.shape, lambda i: (0, 0)),
            pl.BlockSpec((1, nclass), lambda i: (0, 0)),
        ],
        out_specs=pl.BlockSpec((1, nclass), lambda i: (0, 0)),
        out_shape=jax.ShapeDtypeStruct((1, nclass), jnp.float32),
        scratch_shapes=[pltpu.VMEM((1, 3 * D), jnp.float32)],
    )(y, q1a, q1b, q2a, q2b, norm, perm, W2, b2, Wc, bc)


def kernel(h, edge_index, perm_features, W1, b1, W2, b2, Wc, bc):
    src = edge_index[0].astype(jnp.int32)
    dst = edge_index[1].astype(jnp.int32)
    pad = E_PAD - E
    src3 = jnp.concatenate([src, jnp.zeros((pad,), jnp.int32)]).reshape(NW, NB, B)
    dst3 = jnp.concatenate([dst, jnp.full((pad,), N, jnp.int32)]).reshape(NW, NB, B)

    degp = _segsum(src3, dst3, jnp.ones((N, D), jnp.float32))
    d0 = degp[0, :N, 0:1]
    d1 = degp[1, :N, 0:1]
    norm, s0 = _prep1(d0, d1, h)

    p1 = _segsum(src3, dst3, s0)
    s1 = _comb(p1[0, :N], p1[1, :N], norm)
    p2 = _segsum(src3, dst3, s1)
    y, t0 = _layer1(h, p1[0, :N], p1[1, :N], p2[0, :N], p2[1, :N], norm, W1,
                    b1.reshape(1, D))

    q1 = _segsum(src3, dst3, t0)
    t1 = _comb(q1[0, :N], q1[1, :N], norm)
    q2 = _segsum(src3, dst3, t1)

    return _final(y, q1[0, :N], q1[1, :N], q2[0, :N], q2[1, :N], norm,
                  perm_features, W2, b2.reshape(1, D), Wc, bc.reshape(1, -1))


# speedup vs baseline: 2.8929x; 1.2821x over previous
import jax
import jax.numpy as jnp
from jax import lax
from jax.experimental import pallas as pl
from jax.experimental.pallas import tpu as pltpu
from jax.experimental.pallas import tpu_sc as plsc

N = 10000
E = 320000
D = 128
NC, NS, L = 2, 16, 16
NW = NC * NS
B = 128
NB = 80
E_PAD = NW * NB * B
R_T = 640
N_ACC = NS * R_T
BM = 1000


CH = 16
HB = 64


def _fire(tab_hbm, src_c, j, rows, sem):
    pltpu.async_copy(tab_hbm.at[src_c.at[j, pl.ds(0, HB)]], rows.at[pl.ds(0, HB)], sem)
    pltpu.async_copy(tab_hbm.at[src_c.at[j, pl.ds(HB, HB)]], rows.at[pl.ds(HB, HB)], sem)


def _drain(tab_hbm, src_c, j, rows, sem):
    pltpu.make_async_copy(tab_hbm.at[src_c.at[j, pl.ds(0, HB)]], rows.at[pl.ds(0, HB)], sem).wait()
    pltpu.make_async_copy(tab_hbm.at[src_c.at[j, pl.ds(HB, HB)]], rows.at[pl.ds(HB, HB)], sem).wait()


def _segsum_body(src_hbm, dst_hbm, tab_hbm, out_hbm, acc, src_c, dst_c,
                 rows0, rows1, sem0, sem1):
    cid = lax.axis_index("c")
    sid = lax.axis_index("s")
    wid = cid * NS + sid
    zero = jnp.zeros((L,), jnp.float32)

    def zrow(i, c):
        for k in range(D // L):
            rows0[i, pl.ds(k * L, L)] = zero
        return c

    lax.fori_loop(0, B, zrow, 0)
    r0 = sid * R_T
    for k in range(R_T // B):
        pltpu.sync_copy(rows0, acc.at[pl.ds(r0 + k * B, B)])
    plsc.subcore_barrier()

    bufs = (rows0, rows1)
    sems = (sem0, sem1)

    def chunk(c, carry):
        pltpu.sync_copy(src_hbm.at[wid, pl.ds(c * CH, CH)], src_c)
        pltpu.sync_copy(dst_hbm.at[wid, pl.ds(c * CH, CH)], dst_c)
        _fire(tab_hbm, src_c, 0, rows0, sem0)
        _fire(tab_hbm, src_c, 1, rows1, sem1)

        def pair(g, cc):
            for b in range(2):
                j = 2 * g + b
                _drain(tab_hbm, src_c, j, bufs[b], sems[b])
                pltpu.sync_copy(bufs[b], acc.at[dst_c.at[j]], add=True)

                @pl.when(j + 2 < CH)
                def _():
                    _fire(tab_hbm, src_c, j + 2, bufs[b], sems[b])
            return cc

        lax.fori_loop(0, CH // 2, pair, 0)
        return carry

    lax.fori_loop(0, NB // CH, chunk, 0)
    plsc.subcore_barrier()
    pltpu.sync_copy(acc.at[pl.ds(r0, R_T)], out_hbm.at[cid, pl.ds(r0, R_T)])


_segsum = pl.kernel(
    _segsum_body,
    out_type=jax.ShapeDtypeStruct((NC, N_ACC, D), jnp.float32),
    mesh=plsc.VectorSubcoreMesh(core_axis_name="c", subcore_axis_name="s",
                                num_cores=NC, num_subcores=NS),
    scratch_types=[
        pltpu.VMEM_SHARED((N_ACC, D), jnp.float32),
        pltpu.VMEM((CH, B), jnp.int32),
        pltpu.VMEM((CH, B), jnp.int32),
        pltpu.VMEM((B, D), jnp.float32),
        pltpu.VMEM((B, D), jnp.float32),
        pltpu.SemaphoreType.DMA,
        pltpu.SemaphoreType.DMA,
    ],
)


def _degsum_body(dst_hbm, out_hbm, acc, dst_c, rows0, rows1):
    cid = lax.axis_index("c")
    sid = lax.axis_index("s")
    wid = cid * NS + sid
    zero = jnp.zeros((L,), jnp.float32)
    one = jnp.ones((L,), jnp.float32)

    def fill(i, c):
        for k in range(D // L):
            rows0[i, pl.ds(k * L, L)] = zero
            rows1[i, pl.ds(k * L, L)] = one
        return c

    lax.fori_loop(0, B, fill, 0)
    r0 = sid * R_T
    for k in range(R_T // B):
        pltpu.sync_copy(rows0, acc.at[pl.ds(r0 + k * B, B)])
    plsc.subcore_barrier()

    def chunk(c, carry):
        pltpu.sync_copy(dst_hbm.at[wid, pl.ds(c * CH, CH)], dst_c)

        def step(j, cc):
            pltpu.sync_copy(rows1, acc.at[dst_c.at[j]], add=True)
            return cc

        lax.fori_loop(0, CH, step, 0)
        return carry

    lax.fori_loop(0, NB // CH, chunk, 0)
    plsc.subcore_barrier()
    pltpu.sync_copy(acc.at[pl.ds(r0, R_T)], out_hbm.at[cid, pl.ds(r0, R_T)])


_degsum = pl.kernel(
    _degsum_body,
    out_type=jax.ShapeDtypeStruct((NC, N_ACC, D), jnp.float32),
    mesh=plsc.VectorSubcoreMesh(core_axis_name="c", subcore_axis_name="s",
                                num_cores=NC, num_subcores=NS),
    scratch_types=[
        pltpu.VMEM_SHARED((N_ACC, D), jnp.float32),
        pltpu.VMEM((CH, B), jnp.int32),
        pltpu.VMEM((B, D), jnp.float32),
        pltpu.VMEM((B, D), jnp.float32),
    ],
)


def _prep1_body(d0_ref, d1_ref, h_ref, norm_ref, s0_ref):
    deg = d0_ref[...] + d1_ref[...]
    deg = jnp.where(deg < 1.0, 1.0, deg)
    nrm = lax.rsqrt(deg)
    norm_ref[...] = nrm
    s0_ref[...] = h_ref[...] * nrm


def _prep1(d0, d1, h):
    return pl.pallas_call(
        _prep1_body,
        grid=(N // BM,),
        in_specs=[
            pl.BlockSpec((BM, 1), lambda i: (i, 0)),
            pl.BlockSpec((BM, 1), lambda i: (i, 0)),
            pl.BlockSpec((BM, D), lambda i: (i, 0)),
        ],
        out_specs=[
            pl.BlockSpec((BM, 1), lambda i: (i, 0)),
            pl.BlockSpec((BM, D), lambda i: (i, 0)),
        ],
        out_shape=[
            jax.ShapeDtypeStruct((N, 1), jnp.float32),
            jax.ShapeDtypeStruct((N, D), jnp.float32),
        ],
    )(d0, d1, h)


def _comb_body(pa_ref, pb_ref, norm_ref, out_ref):
    nrm = norm_ref[...]
    out_ref[...] = (nrm * nrm) * (pa_ref[...] + pb_ref[...])


def _comb(pa, pb, norm):
    return pl.pallas_call(
        _comb_body,
        grid=(N // BM,),
        in_specs=[
            pl.BlockSpec((BM, D), lambda i: (i, 0)),
            pl.BlockSpec((BM, D), lambda i: (i, 0)),
            pl.BlockSpec((BM, 1), lambda i: (i, 0)),
        ],
        out_specs=pl.BlockSpec((BM, D), lambda i: (i, 0)),
        out_shape=jax.ShapeDtypeStruct((N, D), jnp.float32),
    )(pa, pb, norm)


def _layer1_body(h_ref, p1a_ref, p1b_ref, p2a_ref, p2b_ref, norm_ref, w_ref,
                 b_ref, y_ref, t0_ref):
    nrm = norm_ref[...]
    x1 = nrm * (p1a_ref[...] + p1b_ref[...])
    x2 = nrm * (p2a_ref[...] + p2b_ref[...])
    acc = jnp.dot(h_ref[...], w_ref[0:D, :], preferred_element_type=jnp.float32)
    acc = acc + jnp.dot(x1, w_ref[D:2 * D, :], preferred_element_type=jnp.float32)
    acc = acc + jnp.dot(x2, w_ref[2 * D:3 * D, :], preferred_element_type=jnp.float32)
    y = jnp.maximum(acc + b_ref[...], 0.0)
    y_ref[...] = y
    t0_ref[...] = y * nrm


def _layer1(h, p1a, p1b, p2a, p2b, norm, W1, b1):
    return pl.pallas_call(
        _layer1_body,
        grid=(N // BM,),
        in_specs=[
            pl.BlockSpec((BM, D), lambda i: (i, 0)),
            pl.BlockSpec((BM, D), lambda i: (i, 0)),
            pl.BlockSpec((BM, D), lambda i: (i, 0)),
            pl.BlockSpec((BM, D), lambda i: (i, 0)),
            pl.BlockSpec((BM, D), lambda i: (i, 0)),
            pl.BlockSpec((BM, 1), lambda i: (i, 0)),
            pl.BlockSpec((3 * D, D), lambda i: (0, 0)),
            pl.BlockSpec((1, D), lambda i: (0, 0)),
        ],
        out_specs=[
            pl.BlockSpec((BM, D), lambda i: (i, 0)),
            pl.BlockSpec((BM, D), lambda i: (i, 0)),
        ],
        out_shape=[
            jax.ShapeDtypeStruct((N, D), jnp.float32),
            jax.ShapeDtypeStruct((N, D), jnp.float32),
        ],
    )(h, p1a, p1b, p2a, p2b, norm, W1, b1)


def _final_body(y_ref, q1a_ref, q1b_ref, q2a_ref, q2b_ref, norm_ref, perm_ref,
                w2_ref, b2_ref, wc_ref, bc_ref, out_ref, acc_ref):
    i = pl.program_id(0)

    @pl.when(i == 0)
    def _():
        acc_ref[...] = jnp.zeros_like(acc_ref)

    nrm = norm_ref[...]
    y = y_ref[...]
    y1 = nrm * (q1a_ref[...] + q1b_ref[...])
    y2 = nrm * (q2a_ref[...] + q2b_ref[...])
    acc_ref[...] += jnp.concatenate(
        [jnp.sum(y, axis=0, keepdims=True),
         jnp.sum(y1, axis=0, keepdims=True),
         jnp.sum(y2, axis=0, keepdims=True)], axis=1)

    @pl.when(i == pl.num_programs(0) - 1)
    def _():
        hg = jnp.dot(acc_ref[...] * jnp.float32(1.0 / N), w2_ref[...],
                     preferred_element_type=jnp.float32) + b2_ref[...]
        comb = jnp.concatenate([hg, perm_ref[...]], axis=1)
        out_ref[...] = jnp.dot(comb, wc_ref[...],
                               preferred_element_type=jnp.float32) + bc_ref[...]


def _final(y, q1a, q1b, q2a, q2b, norm, perm, W2, b2, Wc, bc):
    nclass = Wc.shape[1]
    return pl.pallas_call(
        _final_body,
        grid=(N // BM,),
        in_specs=[
            pl.BlockSpec((BM, D), lambda i: (i, 0)),
            pl.BlockSpec((BM, D), lambda i: (i, 0)),
            pl.BlockSpec((BM, D), lambda i: (i, 0)),
            pl.BlockSpec((BM, D), lambda i: (i, 0)),
            pl.BlockSpec((BM, D), lambda i: (i, 0)),
            pl.BlockSpec((BM, 1), lambda i: (i, 0)),
            pl.BlockSpec(perm.shape, lambda i: (0, 0)),
            pl.BlockSpec((3 * D, D), lambda i: (0, 0)),
            pl.BlockSpec((1, D), lambda i: (0, 0)),
            pl.BlockSpec(Wc.shape, lambda i: (0, 0)),
            pl.BlockSpec((1, nclass), lambda i: (0, 0)),
        ],
        out_specs=pl.BlockSpec((1, nclass), lambda i: (0, 0)),
        out_shape=jax.ShapeDtypeStruct((1, nclass), jnp.float32),
        scratch_shapes=[pltpu.VMEM((1, 3 * D), jnp.float32)],
    )(y, q1a, q1b, q2a, q2b, norm, perm, W2, b2, Wc, bc)


def kernel(h, edge_index, perm_features, W1, b1, W2, b2, Wc, bc):
    src = edge_index[0].astype(jnp.int32)
    dst = edge_index[1].astype(jnp.int32)
    pad = E_PAD - E
    src3 = jnp.concatenate([src, jnp.zeros((pad,), jnp.int32)]).reshape(NW, NB, B)
    dst3 = jnp.concatenate([dst, jnp.full((pad,), N, jnp.int32)]).reshape(NW, NB, B)

    degp = _degsum(dst3)
    d0 = degp[0, :N, 0:1]
    d1 = degp[1, :N, 0:1]
    norm, s0 = _prep1(d0, d1, h)

    p1 = _segsum(src3, dst3, s0)
    s1 = _comb(p1[0, :N], p1[1, :N], norm)
    p2 = _segsum(src3, dst3, s1)
    y, t0 = _layer1(h, p1[0, :N], p1[1, :N], p2[0, :N], p2[1, :N], norm, W1,
                    b1.reshape(1, D))

    q1 = _segsum(src3, dst3, t0)
    t1 = _comb(q1[0, :N], q1[1, :N], norm)
    q2 = _segsum(src3, dst3, t1)

    return _final(y, q1[0, :N], q1[1, :N], q2[0, :N], q2[1, :N], norm,
                  perm_features, W2, b2.reshape(1, D), Wc, bc.reshape(1, -1))


# speedup vs baseline: 4.2870x; 1.4819x over previous
import jax
import jax.numpy as jnp
from jax import lax
from jax.experimental import pallas as pl
from jax.experimental.pallas import tpu as pltpu
from jax.experimental.pallas import tpu_sc as plsc

N = 10000
E = 320000
D = 128
NC, NS, L = 2, 16, 16
NW = NC * NS
B = 128
NB = 80
E_PAD = NW * NB * B
R_T = 640
N_ACC = NS * R_T
BM = 1000


CH = 16
HB = 64
CA = 120
CB = 40


def _fire(tab_hbm, src_c, j, rows, sem):
    pltpu.async_copy(tab_hbm.at[src_c.at[j, pl.ds(0, HB)]], rows.at[pl.ds(0, HB)], sem)
    pltpu.async_copy(tab_hbm.at[src_c.at[j, pl.ds(HB, HB)]], rows.at[pl.ds(HB, HB)], sem)


def _drain(tab_hbm, src_c, j, rows, sem):
    pltpu.make_async_copy(tab_hbm.at[src_c.at[j, pl.ds(0, HB)]], rows.at[pl.ds(0, HB)], sem).wait()
    pltpu.make_async_copy(tab_hbm.at[src_c.at[j, pl.ds(HB, HB)]], rows.at[pl.ds(HB, HB)], sem).wait()


def _segsum_body(src_hbm, dst_hbm, tab_hbm, out_hbm, acc, src_c, dst_c,
                 rows0, rows1, sem0, sem1):
    cid = lax.axis_index("c")
    sid = lax.axis_index("s")
    base = jnp.where(cid == 0, CA * sid, NS * CA + CB * sid)
    nch = jnp.where(cid == 0, CA // CH, CB // CH)
    zero = jnp.zeros((L,), jnp.float32)

    def zrow(i, c):
        for k in range(D // L):
            rows0[i, pl.ds(k * L, L)] = zero
        return c

    lax.fori_loop(0, B, zrow, 0)
    r0 = sid * R_T
    for k in range(R_T // B):
        pltpu.sync_copy(rows0, acc.at[pl.ds(r0 + k * B, B)])
    plsc.subcore_barrier()

    bufs = (rows0, rows1)
    sems = (sem0, sem1)

    def chunk(c, carry):
        pltpu.sync_copy(src_hbm.at[pl.ds(base + c * CH, CH)], src_c)
        pltpu.sync_copy(dst_hbm.at[pl.ds(base + c * CH, CH)], dst_c)
        _fire(tab_hbm, src_c, 0, rows0, sem0)
        _fire(tab_hbm, src_c, 1, rows1, sem1)

        def pair(g, cc):
            for b in range(2):
                j = 2 * g + b
                _drain(tab_hbm, src_c, j, bufs[b], sems[b])
                pltpu.sync_copy(bufs[b], acc.at[dst_c.at[j]], add=True)

                @pl.when(j + 2 < CH)
                def _():
                    _fire(tab_hbm, src_c, j + 2, bufs[b], sems[b])
            return cc

        lax.fori_loop(0, CH // 2, pair, 0)
        return carry

    lax.fori_loop(0, nch, chunk, 0)
    plsc.subcore_barrier()
    pltpu.sync_copy(acc.at[pl.ds(r0, R_T)], out_hbm.at[cid, pl.ds(r0, R_T)])


_segsum = pl.kernel(
    _segsum_body,
    out_type=jax.ShapeDtypeStruct((NC, N_ACC, D), jnp.float32),
    mesh=plsc.VectorSubcoreMesh(core_axis_name="c", subcore_axis_name="s",
                                num_cores=NC, num_subcores=NS),
    scratch_types=[
        pltpu.VMEM_SHARED((N_ACC, D), jnp.float32),
        pltpu.VMEM((CH, B), jnp.int32),
        pltpu.VMEM((CH, B), jnp.int32),
        pltpu.VMEM((B, D), jnp.float32),
        pltpu.VMEM((B, D), jnp.float32),
        pltpu.SemaphoreType.DMA,
        pltpu.SemaphoreType.DMA,
    ],
)


def _degsum_body(dst_hbm, out_hbm, acc, dst_c, rows0, rows1):
    cid = lax.axis_index("c")
    sid = lax.axis_index("s")
    wid = cid * NS + sid
    zero = jnp.zeros((L,), jnp.float32)
    one = jnp.ones((L,), jnp.float32)

    def fill(i, c):
        for k in range(D // L):
            rows0[i, pl.ds(k * L, L)] = zero
            rows1[i, pl.ds(k * L, L)] = one
        return c

    lax.fori_loop(0, B, fill, 0)
    r0 = sid * R_T
    for k in range(R_T // B):
        pltpu.sync_copy(rows0, acc.at[pl.ds(r0 + k * B, B)])
    plsc.subcore_barrier()

    def chunk(c, carry):
        pltpu.sync_copy(dst_hbm.at[pl.ds(wid * NB + c * CH, CH)], dst_c)

        def step(j, cc):
            pltpu.sync_copy(rows1, acc.at[dst_c.at[j]], add=True)
            return cc

        lax.fori_loop(0, CH, step, 0)
        return carry

    lax.fori_loop(0, NB // CH, chunk, 0)
    plsc.subcore_barrier()
    pltpu.sync_copy(acc.at[pl.ds(r0, R_T)], out_hbm.at[cid, pl.ds(r0, R_T)])


_degsum = pl.kernel(
    _degsum_body,
    out_type=jax.ShapeDtypeStruct((NC, N_ACC, D), jnp.float32),
    mesh=plsc.VectorSubcoreMesh(core_axis_name="c", subcore_axis_name="s",
                                num_cores=NC, num_subcores=NS),
    scratch_types=[
        pltpu.VMEM_SHARED((N_ACC, D), jnp.float32),
        pltpu.VMEM((CH, B), jnp.int32),
        pltpu.VMEM((B, D), jnp.float32),
        pltpu.VMEM((B, D), jnp.float32),
    ],
)


def _prep1_body(d0_ref, d1_ref, h_ref, norm_ref, s0_ref):
    deg = d0_ref[...] + d1_ref[...]
    deg = jnp.where(deg < 1.0, 1.0, deg)
    nrm = lax.rsqrt(deg)
    norm_ref[...] = nrm
    s0_ref[...] = h_ref[...] * nrm


def _prep1(d0, d1, h):
    return pl.pallas_call(
        _prep1_body,
        grid=(N // BM,),
        in_specs=[
            pl.BlockSpec((BM, 1), lambda i: (i, 0)),
            pl.BlockSpec((BM, 1), lambda i: (i, 0)),
            pl.BlockSpec((BM, D), lambda i: (i, 0)),
        ],
        out_specs=[
            pl.BlockSpec((BM, 1), lambda i: (i, 0)),
            pl.BlockSpec((BM, D), lambda i: (i, 0)),
        ],
        out_shape=[
            jax.ShapeDtypeStruct((N, 1), jnp.float32),
            jax.ShapeDtypeStruct((N, D), jnp.float32),
        ],
    )(d0, d1, h)


def _comb_body(pa_ref, pb_ref, norm_ref, out_ref):
    nrm = norm_ref[...]
    out_ref[...] = (nrm * nrm) * (pa_ref[...] + pb_ref[...])


def _comb(pa, pb, norm):
    return pl.pallas_call(
        _comb_body,
        grid=(N // BM,),
        in_specs=[
            pl.BlockSpec((BM, D), lambda i: (i, 0)),
            pl.BlockSpec((BM, D), lambda i: (i, 0)),
            pl.BlockSpec((BM, 1), lambda i: (i, 0)),
        ],
        out_specs=pl.BlockSpec((BM, D), lambda i: (i, 0)),
        out_shape=jax.ShapeDtypeStruct((N, D), jnp.float32),
    )(pa, pb, norm)


def _layer1_body(h_ref, p1a_ref, p1b_ref, p2a_ref, p2b_ref, norm_ref, w_ref,
                 b_ref, y_ref, t0_ref):
    nrm = norm_ref[...]
    x1 = nrm * (p1a_ref[...] + p1b_ref[...])
    x2 = nrm * (p2a_ref[...] + p2b_ref[...])
    acc = jnp.dot(h_ref[...], w_ref[0:D, :], preferred_element_type=jnp.float32)
    acc = acc + jnp.dot(x1, w_ref[D:2 * D, :], preferred_element_type=jnp.float32)
    acc = acc + jnp.dot(x2, w_ref[2 * D:3 * D, :], preferred_element_type=jnp.float32)
    y = jnp.maximum(acc + b_ref[...], 0.0)
    y_ref[...] = y
    t0_ref[...] = y * nrm


def _layer1(h, p1a, p1b, p2a, p2b, norm, W1, b1):
    return pl.pallas_call(
        _layer1_body,
        grid=(N // BM,),
        in_specs=[
            pl.BlockSpec((BM, D), lambda i: (i, 0)),
            pl.BlockSpec((BM, D), lambda i: (i, 0)),
            pl.BlockSpec((BM, D), lambda i: (i, 0)),
            pl.BlockSpec((BM, D), lambda i: (i, 0)),
            pl.BlockSpec((BM, D), lambda i: (i, 0)),
            pl.BlockSpec((BM, 1), lambda i: (i, 0)),
            pl.BlockSpec((3 * D, D), lambda i: (0, 0)),
            pl.BlockSpec((1, D), lambda i: (0, 0)),
        ],
        out_specs=[
            pl.BlockSpec((BM, D), lambda i: (i, 0)),
            pl.BlockSpec((BM, D), lambda i: (i, 0)),
        ],
        out_shape=[
            jax.ShapeDtypeStruct((N, D), jnp.float32),
            jax.ShapeDtypeStruct((N, D), jnp.float32),
        ],
    )(h, p1a, p1b, p2a, p2b, norm, W1, b1)


def _final_body(y_ref, q1a_ref, q1b_ref, q2a_ref, q2b_ref, norm_ref, perm_ref,
                w2_ref, b2_ref, wc_ref, bc_ref, out_ref, acc_ref):
    i = pl.program_id(0)

    @pl.when(i == 0)
    def _():
        acc_ref[...] = jnp.zeros_like(acc_ref)

    nrm = norm_ref[...]
    y = y_ref[...]
    y1 = nrm * (q1a_ref[...] + q1b_ref[...])
    y2 = nrm * (q2a_ref[...] + q2b_ref[...])
    acc_ref[...] += jnp.concatenate(
        [jnp.sum(y, axis=0, keepdims=True),
         jnp.sum(y1, axis=0, keepdims=True),
         jnp.sum(y2, axis=0, keepdims=True)], axis=1)

    @pl.when(i == pl.num_programs(0) - 1)
    def _():
        hg = jnp.dot(acc_ref[...] * jnp.float32(1.0 / N), w2_ref[...],
                     preferred_element_type=jnp.float32) + b2_ref[...]
        comb = jnp.concatenate([hg, perm_ref[...]], axis=1)
        out_ref[...] = jnp.dot(comb, wc_ref[...],
                               preferred_element_type=jnp.float32) + bc_ref[...]


def _final(y, q1a, q1b, q2a, q2b, norm, perm, W2, b2, Wc, bc):
    nclass = Wc.shape[1]
    return pl.pallas_call(
        _final_body,
        grid=(N // BM,),
        in_specs=[
            pl.BlockSpec((BM, D), lambda i: (i, 0)),
            pl.BlockSpec((BM, D), lambda i: (i, 0)),
            pl.BlockSpec((BM, D), lambda i: (i, 0)),
            pl.BlockSpec((BM, D), lambda i: (i, 0)),
            pl.BlockSpec((BM, D), lambda i: (i, 0)),
            pl.BlockSpec((BM, 1), lambda i: (i, 0)),
            pl.BlockSpec(perm.shape, lambda i: (0, 0)),
            pl.BlockSpec((3 * D, D), lambda i: (0, 0)),
            pl.BlockSpec((1, D), lambda i: (0, 0)),
            pl.BlockSpec(Wc.shape, lambda i: (0, 0)),
            pl.BlockSpec((1, nclass), lambda i: (0, 0)),
        ],
        out_specs=pl.BlockSpec((1, nclass), lambda i: (0, 0)),
        out_shape=jax.ShapeDtypeStruct((1, nclass), jnp.float32),
        scratch_shapes=[pltpu.VMEM((1, 3 * D), jnp.float32)],
    )(y, q1a, q1b, q2a, q2b, norm, perm, W2, b2, Wc, bc)


def kernel(h, edge_index, perm_features, W1, b1, W2, b2, Wc, bc):
    src = edge_index[0].astype(jnp.int32)
    dst = edge_index[1].astype(jnp.int32)
    pad = E_PAD - E
    src3 = jnp.concatenate([src, jnp.zeros((pad,), jnp.int32)]).reshape(NW * NB, B)
    dst3 = jnp.concatenate([dst, jnp.full((pad,), N, jnp.int32)]).reshape(NW * NB, B)

    degp = _degsum(dst3)
    d0 = degp[0, :N, 0:1]
    d1 = degp[1, :N, 0:1]
    norm, s0 = _prep1(d0, d1, h)

    p1 = _segsum(src3, dst3, s0)
    s1 = _comb(p1[0, :N], p1[1, :N], norm)
    p2 = _segsum(src3, dst3, s1)
    y, t0 = _layer1(h, p1[0, :N], p1[1, :N], p2[0, :N], p2[1, :N], norm, W1,
                    b1.reshape(1, D))

    q1 = _segsum(src3, dst3, t0)
    t1 = _comb(q1[0, :N], q1[1, :N], norm)
    q2 = _segsum(src3, dst3, t1)

    return _final(y, q1[0, :N], q1[1, :N], q2[0, :N], q2[1, :N], norm,
                  perm_features, W2, b2.reshape(1, D), Wc, bc.reshape(1, -1))
